# baseline (speedup 1.0000x reference)
import functools

import jax
import jax.numpy as jnp
from jax.experimental import pallas as pl
from jax.experimental.pallas import tpu as pltpu


def _norm_kernel(adj_ref, dinv_ref, delta_ref, adjb_ref, *, nblocks):
    i = pl.program_id(0)
    blk = adj_ref[...]
    db, n = blk.shape
    base = i * db

    colsum = jnp.sum(blk, axis=0, keepdims=True)

    rows = jax.lax.broadcasted_iota(jnp.int32, (db, n), 0)
    cols = jax.lax.broadcasted_iota(jnp.int32, (db, n), 1)
    diag_mask = cols == rows + base
    diagvals = jnp.sum(jnp.where(diag_mask, blk, 0.0), axis=0, keepdims=True)
    in_range = (cols[:1] >= base) & (cols[:1] < base + db)
    delta = jnp.where(in_range & (diagvals == 0.0), 1.0, 0.0)

    @pl.when(i == 0)
    def _():
        dinv_ref[...] = jnp.zeros_like(dinv_ref)
        delta_ref[...] = jnp.zeros_like(delta_ref)

    dinv_ref[...] += colsum + delta
    delta_ref[...] += delta
    adjb_ref[...] = blk.astype(jnp.bfloat16)

    @pl.when(i == nblocks - 1)
    def _():
        deg = dinv_ref[...]
        dinv_ref[...] = jnp.where(deg > 0.0, jax.lax.rsqrt(deg), 0.0)


def _xw_kernel(x_ref, w_ref, dinv_ref, o_ref):
    acc = jnp.dot(x_ref[...], w_ref[...], preferred_element_type=jnp.float32)
    o_ref[...] = (acc * dinv_ref[...]).astype(jnp.bfloat16)


def _spmm_kernel(adj_ref, p_ref, pj_ref, dinv_ref, delta_ref, b_ref, o_ref,
                 acc_ref, *, ni):
    i = pl.program_id(1)

    @pl.when(i == 0)
    def _():
        acc_ref[...] = jnp.zeros_like(acc_ref)

    acc_ref[...] += jax.lax.dot_general(
        adj_ref[...], p_ref[...], (((0,), (0,)), ((), ())),
        preferred_element_type=jnp.float32)

    @pl.when(i == ni - 1)
    def _():
        corr = delta_ref[...] * pj_ref[...].astype(jnp.float32)
        out = dinv_ref[...] * (acc_ref[...] + corr) + b_ref[...]
        o_ref[...] = jnp.maximum(out, 0.0).astype(o_ref.dtype)


_DB = 512
_IB = 512
_JB = 512


def _layer(adj_b, p, dinv_c, delta_c, b_row, out_dtype):
    n = adj_b.shape[0]
    h = p.shape[1]
    ni = n // _IB
    nj = n // _JB
    return pl.pallas_call(
        functools.partial(_spmm_kernel, ni=ni),
        grid=(nj, ni),
        in_specs=[
            pl.BlockSpec((_IB, _JB), lambda j, i: (i, j)),
            pl.BlockSpec((_IB, h), lambda j, i: (i, 0)),
            pl.BlockSpec((_JB, h), lambda j, i: (j, 0)),
            pl.BlockSpec((_JB, 1), lambda j, i: (j, 0)),
            pl.BlockSpec((_JB, 1), lambda j, i: (j, 0)),
            pl.BlockSpec((1, h), lambda j, i: (0, 0)),
        ],
        out_specs=pl.BlockSpec((_JB, h), lambda j, i: (j, 0)),
        out_shape=jax.ShapeDtypeStruct((n, h), out_dtype),
        scratch_shapes=[pltpu.VMEM((_JB, h), jnp.float32)],
        compiler_params=pltpu.CompilerParams(
            dimension_semantics=("parallel", "arbitrary")),
    )(adj_b, p, p, dinv_c, delta_c, b_row)


def _scaled_gemm(x_b, w_b, dinv_c):
    n, f = x_b.shape
    h = w_b.shape[1]
    return pl.pallas_call(
        _xw_kernel,
        grid=(n // _IB,),
        in_specs=[
            pl.BlockSpec((_IB, f), lambda i: (i, 0)),
            pl.BlockSpec((f, h), lambda i: (0, 0)),
            pl.BlockSpec((_IB, 1), lambda i: (i, 0)),
        ],
        out_specs=pl.BlockSpec((_IB, h), lambda i: (i, 0)),
        out_shape=jax.ShapeDtypeStruct((n, h), jnp.bfloat16),
        compiler_params=pltpu.CompilerParams(
            dimension_semantics=("arbitrary",)),
    )(x_b, w_b, dinv_c)


def kernel(x, adj, W1, b1, W2, b2):
    n = adj.shape[0]
    nb = n // _DB

    dinv_row, delta_row, adj_b = pl.pallas_call(
        functools.partial(_norm_kernel, nblocks=nb),
        grid=(nb,),
        in_specs=[pl.BlockSpec((_DB, n), lambda i: (i, 0))],
        out_specs=[
            pl.BlockSpec((1, n), lambda i: (0, 0)),
            pl.BlockSpec((1, n), lambda i: (0, 0)),
            pl.BlockSpec((_DB, n), lambda i: (i, 0)),
        ],
        out_shape=[
            jax.ShapeDtypeStruct((1, n), jnp.float32),
            jax.ShapeDtypeStruct((1, n), jnp.float32),
            jax.ShapeDtypeStruct((n, n), jnp.bfloat16),
        ],
        compiler_params=pltpu.CompilerParams(
            dimension_semantics=("arbitrary",)),
    )(adj)

    dinv_c = dinv_row.reshape(n, 1)
    delta_c = delta_row.reshape(n, 1)
    b1_row = b1.reshape(1, -1)
    b2_row = b2.reshape(1, -1)

    x_b = x.astype(jnp.bfloat16)
    w1_b = W1.astype(jnp.bfloat16)
    w2_b = W2.astype(jnp.bfloat16)

    p1 = _scaled_gemm(x_b, w1_b, dinv_c)
    h = _layer(adj_b, p1, dinv_c, delta_c, b1_row, jnp.bfloat16)
    p2 = _scaled_gemm(h, w2_b, dinv_c)
    out = _layer(adj_b, p2, dinv_c, delta_c, b2_row, jnp.float32)
    return out


# speedup vs baseline: 1.5710x; 1.5710x over previous
import functools

import jax
import jax.numpy as jnp
from jax.experimental import pallas as pl
from jax.experimental.pallas import tpu as pltpu


def _norm_kernel(adj_ref, dinv_ref, delta_ref, adjt_ref, *, nblocks):
    i = pl.program_id(0)
    blk = adj_ref[...]
    db, n = blk.shape
    base = i * db

    colsum = jnp.sum(blk, axis=0, keepdims=True)

    rows = jax.lax.broadcasted_iota(jnp.int32, (db, n), 0)
    cols = jax.lax.broadcasted_iota(jnp.int32, (db, n), 1)
    diag_mask = cols == rows + base
    diagvals = jnp.sum(jnp.where(diag_mask, blk, 0.0), axis=0, keepdims=True)
    in_range = (cols[:1] >= base) & (cols[:1] < base + db)
    delta = jnp.where(in_range & (diagvals == 0.0), 1.0, 0.0)

    @pl.when(i == 0)
    def _():
        dinv_ref[...] = jnp.zeros_like(dinv_ref)
        delta_ref[...] = jnp.zeros_like(delta_ref)

    dinv_ref[...] += colsum + delta
    delta_ref[...] += delta
    adjt_ref[...] = blk.astype(jnp.bfloat16).T

    @pl.when(i == nblocks - 1)
    def _():
        deg = dinv_ref[...]
        dinv_ref[...] = jnp.where(deg > 0.0, jax.lax.rsqrt(deg), 0.0)


def _xw_kernel(x_ref, w_ref, dinv_ref, o_ref):
    acc = jnp.dot(x_ref[...], w_ref[...], preferred_element_type=jnp.float32)
    o_ref[...] = (acc * dinv_ref[...]).astype(jnp.bfloat16)


def _spmm_kernel(adjt_ref, p_ref, pj_ref, dinv_ref, delta_ref, b_ref, o_ref):
    acc = jnp.dot(adjt_ref[...], p_ref[...],
                  preferred_element_type=jnp.float32)
    corr = delta_ref[...] * pj_ref[...].astype(jnp.float32)
    out = dinv_ref[...] * (acc + corr) + b_ref[...]
    o_ref[...] = jnp.maximum(out, 0.0).astype(o_ref.dtype)


_DB = 512
_IB = 512
_JB = 512


def _layer(adj_t, p, dinv_c, delta_c, b_row, out_dtype):
    n = adj_t.shape[0]
    h = p.shape[1]
    return pl.pallas_call(
        _spmm_kernel,
        grid=(n // _JB,),
        in_specs=[
            pl.BlockSpec((_JB, n), lambda j: (j, 0)),
            pl.BlockSpec((n, h), lambda j: (0, 0)),
            pl.BlockSpec((_JB, h), lambda j: (j, 0)),
            pl.BlockSpec((_JB, 1), lambda j: (j, 0)),
            pl.BlockSpec((_JB, 1), lambda j: (j, 0)),
            pl.BlockSpec((1, h), lambda j: (0, 0)),
        ],
        out_specs=pl.BlockSpec((_JB, h), lambda j: (j, 0)),
        out_shape=jax.ShapeDtypeStruct((n, h), out_dtype),
        compiler_params=pltpu.CompilerParams(
            dimension_semantics=("parallel",)),
    )(adj_t, p, p, dinv_c, delta_c, b_row)


def _scaled_gemm(x_b, w_b, dinv_c):
    n, f = x_b.shape
    h = w_b.shape[1]
    return pl.pallas_call(
        _xw_kernel,
        grid=(n // _IB,),
        in_specs=[
            pl.BlockSpec((_IB, f), lambda i: (i, 0)),
            pl.BlockSpec((f, h), lambda i: (0, 0)),
            pl.BlockSpec((_IB, 1), lambda i: (i, 0)),
        ],
        out_specs=pl.BlockSpec((_IB, h), lambda i: (i, 0)),
        out_shape=jax.ShapeDtypeStruct((n, h), jnp.bfloat16),
        compiler_params=pltpu.CompilerParams(
            dimension_semantics=("parallel",)),
    )(x_b, w_b, dinv_c)


def kernel(x, adj, W1, b1, W2, b2):
    n = adj.shape[0]
    nb = n // _DB

    dinv_row, delta_row, adj_t = pl.pallas_call(
        functools.partial(_norm_kernel, nblocks=nb),
        grid=(nb,),
        in_specs=[pl.BlockSpec((_DB, n), lambda i: (i, 0))],
        out_specs=[
            pl.BlockSpec((1, n), lambda i: (0, 0)),
            pl.BlockSpec((1, n), lambda i: (0, 0)),
            pl.BlockSpec((n, _DB), lambda i: (0, i)),
        ],
        out_shape=[
            jax.ShapeDtypeStruct((1, n), jnp.float32),
            jax.ShapeDtypeStruct((1, n), jnp.float32),
            jax.ShapeDtypeStruct((n, n), jnp.bfloat16),
        ],
        compiler_params=pltpu.CompilerParams(
            dimension_semantics=("arbitrary",)),
    )(adj)

    dinv_c = dinv_row.reshape(n, 1)
    delta_c = delta_row.reshape(n, 1)
    b1_row = b1.reshape(1, -1)
    b2_row = b2.reshape(1, -1)

    x_b = x.astype(jnp.bfloat16)
    w1_b = W1.astype(jnp.bfloat16)
    w2_b = W2.astype(jnp.bfloat16)

    p1 = _scaled_gemm(x_b, w1_b, dinv_c)
    h = _layer(adj_t, p1, dinv_c, delta_c, b1_row, jnp.bfloat16)
    p2 = _scaled_gemm(h, w2_b, dinv_c)
    out = _layer(adj_t, p2, dinv_c, delta_c, b2_row, jnp.float32)
    return out


# speedup vs baseline: 1.5941x; 1.0147x over previous
import functools

import jax
import jax.numpy as jnp
from jax.experimental import pallas as pl
from jax.experimental.pallas import tpu as pltpu


def _norm_kernel(adj_ref, dinv_ref, delta_ref, adjt_ref, *, nblocks):
    i = pl.program_id(0)
    blk = adj_ref[...]
    db, n = blk.shape
    base = i * db

    colsum = jnp.sum(blk, axis=0, keepdims=True)

    rows = jax.lax.broadcasted_iota(jnp.int32, (db, n), 0)
    cols = jax.lax.broadcasted_iota(jnp.int32, (db, n), 1)
    diag_mask = cols == rows + base
    diagvals = jnp.sum(jnp.where(diag_mask, blk, 0.0), axis=0, keepdims=True)
    in_range = (cols[:1] >= base) & (cols[:1] < base + db)
    delta = jnp.where(in_range & (diagvals == 0.0), 1.0, 0.0)

    @pl.when(i == 0)
    def _():
        dinv_ref[...] = jnp.zeros_like(dinv_ref)
        delta_ref[...] = jnp.zeros_like(delta_ref)

    dinv_ref[...] += colsum + delta
    delta_ref[...] += delta
    adjt_ref[...] = blk.astype(jnp.bfloat16).T

    @pl.when(i == nblocks - 1)
    def _():
        deg = dinv_ref[...]
        dinv_ref[...] = jnp.where(deg > 0.0, jax.lax.rsqrt(deg), 0.0)


def _xw_kernel(x_ref, w_ref, dinv_ref, o_ref):
    acc = jnp.dot(x_ref[...], w_ref[...], preferred_element_type=jnp.float32)
    o_ref[...] = (acc * dinv_ref[...]).astype(jnp.bfloat16)


def _spmm_kernel(adjt_ref, p_ref, pj_ref, dinv_ref, delta_ref, b_ref, o_ref):
    acc = jnp.dot(adjt_ref[...], p_ref[...],
                  preferred_element_type=jnp.float32)
    corr = delta_ref[...] * pj_ref[...].astype(jnp.float32)
    out = dinv_ref[...] * (acc + corr) + b_ref[...]
    o_ref[...] = jnp.maximum(out, 0.0).astype(o_ref.dtype)


_DB = 512
_IB = 1024
_JB = 1024


def _layer(adj_t, p, dinv_c, delta_c, b_row, out_dtype):
    n = adj_t.shape[0]
    h = p.shape[1]
    return pl.pallas_call(
        _spmm_kernel,
        grid=(n // _JB,),
        in_specs=[
            pl.BlockSpec((_JB, n), lambda j: (j, 0)),
            pl.BlockSpec((n, h), lambda j: (0, 0)),
            pl.BlockSpec((_JB, h), lambda j: (j, 0)),
            pl.BlockSpec((_JB, 1), lambda j: (j, 0)),
            pl.BlockSpec((_JB, 1), lambda j: (j, 0)),
            pl.BlockSpec((1, h), lambda j: (0, 0)),
        ],
        out_specs=pl.BlockSpec((_JB, h), lambda j: (j, 0)),
        out_shape=jax.ShapeDtypeStruct((n, h), out_dtype),
        compiler_params=pltpu.CompilerParams(
            dimension_semantics=("parallel",)),
    )(adj_t, p, p, dinv_c, delta_c, b_row)


def _scaled_gemm(x_b, w_b, dinv_c):
    n, f = x_b.shape
    h = w_b.shape[1]
    return pl.pallas_call(
        _xw_kernel,
        grid=(n // _IB,),
        in_specs=[
            pl.BlockSpec((_IB, f), lambda i: (i, 0)),
            pl.BlockSpec((f, h), lambda i: (0, 0)),
            pl.BlockSpec((_IB, 1), lambda i: (i, 0)),
        ],
        out_specs=pl.BlockSpec((_IB, h), lambda i: (i, 0)),
        out_shape=jax.ShapeDtypeStruct((n, h), jnp.bfloat16),
        compiler_params=pltpu.CompilerParams(
            dimension_semantics=("parallel",)),
    )(x_b, w_b, dinv_c)


def kernel(x, adj, W1, b1, W2, b2):
    n = adj.shape[0]
    nb = n // _DB

    dinv_row, delta_row, adj_t = pl.pallas_call(
        functools.partial(_norm_kernel, nblocks=nb),
        grid=(nb,),
        in_specs=[pl.BlockSpec((_DB, n), lambda i: (i, 0))],
        out_specs=[
            pl.BlockSpec((1, n), lambda i: (0, 0)),
            pl.BlockSpec((1, n), lambda i: (0, 0)),
            pl.BlockSpec((n, _DB), lambda i: (0, i)),
        ],
        out_shape=[
            jax.ShapeDtypeStruct((1, n), jnp.float32),
            jax.ShapeDtypeStruct((1, n), jnp.float32),
            jax.ShapeDtypeStruct((n, n), jnp.bfloat16),
        ],
        compiler_params=pltpu.CompilerParams(
            dimension_semantics=("arbitrary",)),
    )(adj)

    dinv_c = dinv_row.reshape(n, 1)
    delta_c = delta_row.reshape(n, 1)
    b1_row = b1.reshape(1, -1)
    b2_row = b2.reshape(1, -1)

    x_b = x.astype(jnp.bfloat16)
    w1_b = W1.astype(jnp.bfloat16)
    w2_b = W2.astype(jnp.bfloat16)

    p1 = _scaled_gemm(x_b, w1_b, dinv_c)
    h = _layer(adj_t, p1, dinv_c, delta_c, b1_row, jnp.bfloat16)
    p2 = _scaled_gemm(h, w2_b, dinv_c)
    out = _layer(adj_t, p2, dinv_c, delta_c, b2_row, jnp.float32)
    return out


# speedup vs baseline: 1.7908x; 1.1234x over previous
import functools

import jax
import jax.numpy as jnp
from jax.experimental import pallas as pl
from jax.experimental.pallas import tpu as pltpu


def _norm_gemm_kernel(adj_ref, x_ref, w1_ref, dinv_ref, delta_ref,
                      adjb_ref, p1_ref, *, nblocks):
    i = pl.program_id(0)
    blk = adj_ref[...]
    db, n = blk.shape
    base = i * db

    colsum = jnp.sum(blk, axis=0, keepdims=True)

    rows = jax.lax.broadcasted_iota(jnp.int32, (db, n), 0)
    cols = jax.lax.broadcasted_iota(jnp.int32, (db, n), 1)
    diag_mask = cols == rows + base
    diagvals = jnp.sum(jnp.where(diag_mask, blk, 0.0), axis=0, keepdims=True)
    in_range = (cols[:1] >= base) & (cols[:1] < base + db)
    delta = jnp.where(in_range & (diagvals == 0.0), 1.0, 0.0)

    @pl.when(i == 0)
    def _():
        dinv_ref[...] = jnp.zeros_like(dinv_ref)
        delta_ref[...] = jnp.zeros_like(delta_ref)

    dinv_ref[...] += colsum + delta
    delta_ref[...] += delta
    adjb_ref[...] = blk.astype(jnp.bfloat16)

    p1_ref[...] = jnp.dot(x_ref[...], w1_ref[...],
                          preferred_element_type=jnp.float32
                          ).astype(jnp.bfloat16)

    @pl.when(i == nblocks - 1)
    def _():
        deg = dinv_ref[...]
        dinv_ref[...] = jnp.where(deg > 0.0, jax.lax.rsqrt(deg), 0.0)


def _layer1_kernel(adjb_ref, p1_ref, dinv_ref, delta_ref, b1_ref, w2_ref,
                   p2t_ref, acc_ref, pts_ref, *, ni, ib):
    i = pl.program_id(0)

    @pl.when(i == 0)
    def _():
        acc_ref[...] = jnp.zeros_like(acc_ref)

    @pl.when(i < ni)
    def _():
        sl = pl.ds(i * ib, ib)
        pt = p1_ref[sl, :].T
        pts = (pt.astype(jnp.float32) * dinv_ref[:, sl]).astype(jnp.bfloat16)
        pts_ref[:, sl] = pts
        acc_ref[...] += jnp.dot(pts, adjb_ref[...],
                                preferred_element_type=jnp.float32)

    @pl.when(i == ni)
    def _():
        corr = delta_ref[...] * pts_ref[...].astype(jnp.float32)
        ht = jnp.maximum(dinv_ref[...] * (acc_ref[...] + corr) + b1_ref[...],
                         0.0).astype(jnp.bfloat16)
        p2t_ref[...] = jnp.dot(w2_ref[...].T, ht,
                               preferred_element_type=jnp.float32
                               ).astype(jnp.bfloat16)


def _layer2_kernel(adjb_ref, p2t_ref, dinv_ref, delta_ref, b2_ref,
                   o_ref, acc_ref, *, ni, ib):
    i = pl.program_id(0)

    @pl.when(i == 0)
    def _():
        acc_ref[...] = jnp.zeros_like(acc_ref)

    @pl.when(i < ni)
    def _():
        sl = pl.ds(i * ib, ib)
        pts = (p2t_ref[:, sl].astype(jnp.float32)
               * dinv_ref[:, sl]).astype(jnp.bfloat16)
        acc_ref[...] += jnp.dot(pts, adjb_ref[...],
                                preferred_element_type=jnp.float32)

    @pl.when(i == ni)
    def _():
        p2ts = p2t_ref[...].astype(jnp.float32) * dinv_ref[...]
        out_t = dinv_ref[...] * (acc_ref[...] + delta_ref[...] * p2ts) \
            + b2_ref[...]
        o_ref[...] = jnp.maximum(out_t, 0.0).T


_DB = 512
_IB = 1024


def kernel(x, adj, W1, b1, W2, b2):
    n = adj.shape[0]
    f = x.shape[1]
    h1 = W1.shape[1]
    h2 = W2.shape[1]
    nb = n // _DB
    ni = n // _IB

    x_b = x.astype(jnp.bfloat16)
    w1_b = W1.astype(jnp.bfloat16)
    w2_b = W2.astype(jnp.bfloat16)
    b1_c = b1.reshape(-1, 1)
    b2_c = b2.reshape(-1, 1)

    dinv_row, delta_row, adj_b, p1 = pl.pallas_call(
        functools.partial(_norm_gemm_kernel, nblocks=nb),
        grid=(nb,),
        in_specs=[
            pl.BlockSpec((_DB, n), lambda i: (i, 0)),
            pl.BlockSpec((_DB, f), lambda i: (i, 0)),
            pl.BlockSpec((f, h1), lambda i: (0, 0)),
        ],
        out_specs=[
            pl.BlockSpec((1, n), lambda i: (0, 0)),
            pl.BlockSpec((1, n), lambda i: (0, 0)),
            pl.BlockSpec((_DB, n), lambda i: (i, 0)),
            pl.BlockSpec((_DB, h1), lambda i: (i, 0)),
        ],
        out_shape=[
            jax.ShapeDtypeStruct((1, n), jnp.float32),
            jax.ShapeDtypeStruct((1, n), jnp.float32),
            jax.ShapeDtypeStruct((n, n), jnp.bfloat16),
            jax.ShapeDtypeStruct((n, h1), jnp.bfloat16),
        ],
        compiler_params=pltpu.CompilerParams(
            dimension_semantics=("arbitrary",)),
    )(adj, x_b, w1_b)

    adj_spec = pl.BlockSpec((_IB, n), lambda i: (jnp.minimum(i, ni - 1), 0))

    p2t = pl.pallas_call(
        functools.partial(_layer1_kernel, ni=ni, ib=_IB),
        grid=(ni + 1,),
        in_specs=[
            adj_spec,
            pl.BlockSpec((n, h1), lambda i: (0, 0)),
            pl.BlockSpec((1, n), lambda i: (0, 0)),
            pl.BlockSpec((1, n), lambda i: (0, 0)),
            pl.BlockSpec((h1, 1), lambda i: (0, 0)),
            pl.BlockSpec((h1, h2), lambda i: (0, 0)),
        ],
        out_specs=pl.BlockSpec((h2, n), lambda i: (0, 0)),
        out_shape=jax.ShapeDtypeStruct((h2, n), jnp.bfloat16),
        scratch_shapes=[
            pltpu.VMEM((h1, n), jnp.float32),
            pltpu.VMEM((h1, n), jnp.bfloat16),
        ],
        compiler_params=pltpu.CompilerParams(
            dimension_semantics=("arbitrary",)),
    )(adj_b, p1, dinv_row, delta_row, b1_c, w2_b)

    out = pl.pallas_call(
        functools.partial(_layer2_kernel, ni=ni, ib=_IB),
        grid=(ni + 1,),
        in_specs=[
            adj_spec,
            pl.BlockSpec((h2, n), lambda i: (0, 0)),
            pl.BlockSpec((1, n), lambda i: (0, 0)),
            pl.BlockSpec((1, n), lambda i: (0, 0)),
            pl.BlockSpec((h2, 1), lambda i: (0, 0)),
        ],
        out_specs=pl.BlockSpec((n, h2), lambda i: (0, 0)),
        out_shape=jax.ShapeDtypeStruct((n, h2), jnp.float32),
        scratch_shapes=[pltpu.VMEM((h2, n), jnp.float32)],
        compiler_params=pltpu.CompilerParams(
            dimension_semantics=("arbitrary",)),
    )(adj_b, p2t, dinv_row, delta_row, b2_c)

    return out


# speedup vs baseline: 1.9870x; 1.1095x over previous
import functools

import jax
import jax.numpy as jnp
from jax.experimental import pallas as pl
from jax.experimental.pallas import tpu as pltpu


def _norm_gemm_kernel(adj_ref, x_ref, w1_ref, dinv_ref, dinvb_ref,
                      delta_ref, adjb_ref, p1t_ref, *, nblocks):
    i = pl.program_id(0)
    blk = adj_ref[...]
    db, n = blk.shape
    base = i * db

    colsum = jnp.sum(blk, axis=0, keepdims=True)

    rows = jax.lax.broadcasted_iota(jnp.int32, (db, n), 0)
    cols = jax.lax.broadcasted_iota(jnp.int32, (db, n), 1)
    diag_mask = cols == rows + base
    diagvals = jnp.sum(jnp.where(diag_mask, blk, 0.0), axis=0, keepdims=True)
    in_range = (cols[:1] >= base) & (cols[:1] < base + db)
    delta = jnp.where(in_range & (diagvals == 0.0), 1.0, 0.0)

    @pl.when(i == 0)
    def _():
        dinv_ref[...] = jnp.zeros_like(dinv_ref)
        delta_ref[...] = jnp.zeros_like(delta_ref)

    dinv_ref[...] += colsum + delta
    delta_ref[...] += delta
    adjb_ref[...] = blk.astype(jnp.bfloat16)

    p1_blk = jnp.dot(x_ref[...].astype(jnp.bfloat16), w1_ref[...],
                     preferred_element_type=jnp.float32)
    p1t_ref[...] = p1_blk.astype(jnp.bfloat16).T

    @pl.when(i == nblocks - 1)
    def _():
        deg = dinv_ref[...]
        dinv = jnp.where(deg > 0.0, jax.lax.rsqrt(deg), 0.0)
        dinv_ref[...] = dinv
        dinvb_ref[...] = dinv.astype(jnp.bfloat16)


def _layers_kernel(adjb_ref, p1t_ref, dinv_ref, dinvb_ref, delta_ref,
                   b1_ref, b2_ref, w2_ref, o_ref, acc_ref, p2t_ref,
                   *, ni, ib):
    i = pl.program_id(0)

    @pl.when(i == 0)
    def _():
        acc_ref[...] = jnp.zeros_like(acc_ref)

    @pl.when(i < ni)
    def _():
        sl = pl.ds(i * ib, ib)
        pts = p1t_ref[:, sl] * dinvb_ref[:, sl]
        acc_ref[...] += jnp.dot(pts, adjb_ref[...],
                                preferred_element_type=jnp.float32)

    @pl.when(i == ni)
    def _():
        p1ts = (p1t_ref[...] * dinvb_ref[...]).astype(jnp.float32)
        corr = delta_ref[...] * p1ts
        ht = jnp.maximum(dinv_ref[...] * (acc_ref[...] + corr) + b1_ref[...],
                         0.0).astype(jnp.bfloat16)
        p2t_ref[...] = jnp.dot(w2_ref[...].T, ht,
                               preferred_element_type=jnp.float32
                               ).astype(jnp.bfloat16)
        acc_ref[...] = jnp.zeros_like(acc_ref)

    @pl.when((i > ni) & (i < 2 * ni + 1))
    def _():
        j = i - ni - 1
        sl = pl.ds(j * ib, ib)
        pts = p2t_ref[:, sl] * dinvb_ref[:, sl]
        acc_ref[...] += jnp.dot(pts, adjb_ref[...],
                                preferred_element_type=jnp.float32)

    @pl.when(i == 2 * ni + 1)
    def _():
        p2ts = (p2t_ref[...] * dinvb_ref[...]).astype(jnp.float32)
        out_t = dinv_ref[...] * (acc_ref[...] + delta_ref[...] * p2ts) \
            + b2_ref[...]
        o_ref[...] = jnp.maximum(out_t, 0.0).T


_DB = 512
_IB = 1024


def kernel(x, adj, W1, b1, W2, b2):
    n = adj.shape[0]
    f = x.shape[1]
    h1 = W1.shape[1]
    h2 = W2.shape[1]
    nb = n // _DB
    ni = n // _IB

    w1_b = W1.astype(jnp.bfloat16)
    w2_b = W2.astype(jnp.bfloat16)
    b1_c = b1.reshape(-1, 1)
    b2_c = b2.reshape(-1, 1)

    dinv_row, dinvb_row, delta_row, adj_b, p1t = pl.pallas_call(
        functools.partial(_norm_gemm_kernel, nblocks=nb),
        grid=(nb,),
        in_specs=[
            pl.BlockSpec((_DB, n), lambda i: (i, 0)),
            pl.BlockSpec((_DB, f), lambda i: (i, 0)),
            pl.BlockSpec((f, h1), lambda i: (0, 0)),
        ],
        out_specs=[
            pl.BlockSpec((1, n), lambda i: (0, 0)),
            pl.BlockSpec((1, n), lambda i: (0, 0)),
            pl.BlockSpec((1, n), lambda i: (0, 0)),
            pl.BlockSpec((_DB, n), lambda i: (i, 0)),
            pl.BlockSpec((h1, _DB), lambda i: (0, i)),
        ],
        out_shape=[
            jax.ShapeDtypeStruct((1, n), jnp.float32),
            jax.ShapeDtypeStruct((1, n), jnp.bfloat16),
            jax.ShapeDtypeStruct((1, n), jnp.float32),
            jax.ShapeDtypeStruct((n, n), jnp.bfloat16),
            jax.ShapeDtypeStruct((h1, n), jnp.bfloat16),
        ],
        compiler_params=pltpu.CompilerParams(
            dimension_semantics=("arbitrary",)),
    )(adj, x, w1_b)

    out = pl.pallas_call(
        functools.partial(_layers_kernel, ni=ni, ib=_IB),
        grid=(2 * ni + 2,),
        in_specs=[
            pl.BlockSpec(
                (_IB, n),
                lambda i: (jnp.minimum(
                    jnp.where(i <= ni, i, i - ni - 1), ni - 1), 0)),
            pl.BlockSpec((h1, n), lambda i: (0, 0)),
            pl.BlockSpec((1, n), lambda i: (0, 0)),
            pl.BlockSpec((1, n), lambda i: (0, 0)),
            pl.BlockSpec((1, n), lambda i: (0, 0)),
            pl.BlockSpec((h1, 1), lambda i: (0, 0)),
            pl.BlockSpec((h2, 1), lambda i: (0, 0)),
            pl.BlockSpec((h1, h2), lambda i: (0, 0)),
        ],
        out_specs=pl.BlockSpec((n, h2), lambda i: (0, 0)),
        out_shape=jax.ShapeDtypeStruct((n, h2), jnp.float32),
        scratch_shapes=[
            pltpu.VMEM((h1, n), jnp.float32),
            pltpu.VMEM((h2, n), jnp.bfloat16),
        ],
        compiler_params=pltpu.CompilerParams(
            dimension_semantics=("arbitrary",)),
    )(adj_b, p1t, dinv_row, dinvb_row, delta_row, b1_c, b2_c, w2_b)

    return out


# speedup vs baseline: 2.0312x; 1.0222x over previous
import functools

import jax
import jax.numpy as jnp
from jax.experimental import pallas as pl
from jax.experimental.pallas import tpu as pltpu


def _gcn_kernel(adj_ref, x_ref, w1_ref, w2_ref, b1_ref, b2_ref, o_ref,
                adjb_ref, p1t_ref, p2t_ref, acc_ref, deg_ref, dinvb_ref,
                delta_ref, *, nb, db, ni, ib, ob):
    i = pl.program_id(0)
    s_acc1 = nb
    s_epi1 = nb + ni
    s_acc2 = nb + ni + 1
    s_epi2 = nb + 2 * ni + 1

    @pl.when(i == 0)
    def _():
        deg_ref[...] = jnp.zeros_like(deg_ref)
        delta_ref[...] = jnp.zeros_like(delta_ref)
        acc_ref[...] = jnp.zeros_like(acc_ref)

    @pl.when(i < nb)
    def _():
        blk = adj_ref[...]
        n = blk.shape[1]
        base = i * db

        deg_row = jnp.sum(blk, axis=0, keepdims=True)

        rows = jax.lax.broadcasted_iota(jnp.int32, (db, n), 0)
        cols = jax.lax.broadcasted_iota(jnp.int32, (db, n), 1)
        diagvals = jnp.sum(jnp.where(cols == rows + base, blk, 0.0),
                           axis=0, keepdims=True)
        in_range = (cols[:1] >= base) & (cols[:1] < base + db)
        delta = jnp.where(in_range & (diagvals == 0.0), 1.0, 0.0)

        deg_ref[...] += deg_row + delta
        delta_ref[...] += delta
        adjb_ref[pl.ds(base, db), :] = blk.astype(jnp.bfloat16)

        p1_blk = jnp.dot(x_ref[...].astype(jnp.bfloat16), w1_ref[...],
                         preferred_element_type=jnp.float32)
        p1t_ref[:, pl.ds(base, db)] = p1_blk.astype(jnp.bfloat16).T

        @pl.when(i == nb - 1)
        def _():
            deg = deg_ref[...]
            dinv = jnp.where(deg > 0.0, jax.lax.rsqrt(deg), 0.0)
            deg_ref[...] = dinv
            dinvb_ref[...] = dinv.astype(jnp.bfloat16)

    @pl.when((i >= s_acc1) & (i < s_epi1))
    def _():
        sl = pl.ds((i - s_acc1) * ib, ib)
        pts = p1t_ref[:, sl] * dinvb_ref[:, sl]
        acc_ref[...] += jnp.dot(pts, adjb_ref[pl.ds((i - s_acc1) * ib, ib), :],
                                preferred_element_type=jnp.float32)

    @pl.when(i == s_epi1)
    def _():
        p1ts = (p1t_ref[...] * dinvb_ref[...]).astype(jnp.float32)
        corr = delta_ref[...] * p1ts
        ht = jnp.maximum(deg_ref[...] * (acc_ref[...] + corr) + b1_ref[...],
                         0.0).astype(jnp.bfloat16)
        p2t_ref[...] = jnp.dot(w2_ref[...].T, ht,
                               preferred_element_type=jnp.float32
                               ).astype(jnp.bfloat16)
        acc_ref[...] = jnp.zeros_like(acc_ref)

    @pl.when((i >= s_acc2) & (i < s_epi2))
    def _():
        sl = pl.ds((i - s_acc2) * ib, ib)
        pts = p2t_ref[:, sl] * dinvb_ref[:, sl]
        acc_ref[...] += jnp.dot(pts, adjb_ref[pl.ds((i - s_acc2) * ib, ib), :],
                                preferred_element_type=jnp.float32)

    @pl.when(i >= s_epi2)
    def _():
        sl = pl.ds((i - s_epi2) * ob, ob)
        p2ts = (p2t_ref[:, sl] * dinvb_ref[:, sl]).astype(jnp.float32)
        out_t = deg_ref[:, sl] * (acc_ref[:, sl] + delta_ref[:, sl] * p2ts) \
            + b2_ref[...]
        o_ref[...] = jnp.maximum(out_t, 0.0).T


_DB = 128
_IB = 1024
_OB = 512


def kernel(x, adj, W1, b1, W2, b2):
    n = adj.shape[0]
    f = x.shape[1]
    h1 = W1.shape[1]
    h2 = W2.shape[1]
    nb = n // _DB
    ni = n // _IB
    no = n // _OB
    nsteps = nb + 2 * ni + 1 + no

    w1_b = W1.astype(jnp.bfloat16)
    w2_b = W2.astype(jnp.bfloat16)
    b1_c = b1.reshape(-1, 1)
    b2_c = b2.reshape(-1, 1)

    s_epi2 = nb + 2 * ni + 1

    out = pl.pallas_call(
        functools.partial(_gcn_kernel, nb=nb, db=_DB, ni=ni, ib=_IB, ob=_OB),
        grid=(nsteps,),
        in_specs=[
            pl.BlockSpec((_DB, n), lambda i: (jnp.minimum(i, nb - 1), 0)),
            pl.BlockSpec((_DB, f), lambda i: (jnp.minimum(i, nb - 1), 0)),
            pl.BlockSpec((f, h1), lambda i: (0, 0)),
            pl.BlockSpec((h1, h2), lambda i: (0, 0)),
            pl.BlockSpec((h1, 1), lambda i: (0, 0)),
            pl.BlockSpec((h2, 1), lambda i: (0, 0)),
        ],
        out_specs=pl.BlockSpec(
            (_OB, h2), lambda i: (jnp.clip(i - s_epi2, 0, no - 1), 0)),
        out_shape=jax.ShapeDtypeStruct((n, h2), jnp.float32),
        scratch_shapes=[
            pltpu.VMEM((n, n), jnp.bfloat16),
            pltpu.VMEM((h1, n), jnp.bfloat16),
            pltpu.VMEM((h2, n), jnp.bfloat16),
            pltpu.VMEM((h1, n), jnp.float32),
            pltpu.VMEM((1, n), jnp.float32),
            pltpu.VMEM((1, n), jnp.bfloat16),
            pltpu.VMEM((1, n), jnp.float32),
        ],
        compiler_params=pltpu.CompilerParams(
            dimension_semantics=("arbitrary",)),
    )(adj, x, w1_b, w2_b, b1_c, b2_c)

    return out


# speedup vs baseline: 2.2816x; 1.1233x over previous
import functools

import jax
import jax.numpy as jnp
from jax.experimental import pallas as pl
from jax.experimental.pallas import tpu as pltpu


def _gcn_kernel(adj_ref, x_ref, w1_ref, w2_ref, b1_ref, b2_ref, o_ref,
                adjb_ref, p1t_ref, p2t_ref, acc_ref, deg_ref, dinvb_ref,
                delta_ref, *, nb, db, ob):
    i = pl.program_id(0)
    s_l1 = nb
    s_l2 = nb + 1
    s_w = nb + 2

    @pl.when(i == 0)
    def _():
        deg_ref[...] = jnp.zeros_like(deg_ref)

    @pl.when(i < nb)
    def _():
        blk = adj_ref[...]
        base = i * db

        deg_ref[...] += jnp.sum(blk, axis=0, keepdims=True)

        dblk = adj_ref[:, pl.ds(base, db)]
        rows = jax.lax.broadcasted_iota(jnp.int32, (db, db), 0)
        cols = jax.lax.broadcasted_iota(jnp.int32, (db, db), 1)
        diagvals = jnp.sum(jnp.where(rows == cols, dblk, 0.0),
                           axis=0, keepdims=True)
        delta_blk = jnp.where(diagvals == 0.0, 1.0, 0.0)
        delta_ref[:, pl.ds(base, db)] = delta_blk

        adjb_ref[pl.ds(base, db), :] = blk.astype(jnp.bfloat16)

        p1_blk = jnp.dot(x_ref[...].astype(jnp.bfloat16), w1_ref[...],
                         preferred_element_type=jnp.float32)
        p1t_ref[:, pl.ds(base, db)] = p1_blk.astype(jnp.bfloat16).T

        @pl.when(i == nb - 1)
        def _():
            deg = deg_ref[...] + delta_ref[...]
            dinv = jnp.where(deg > 0.0, jax.lax.rsqrt(deg), 0.0)
            deg_ref[...] = dinv
            dinvb_ref[...] = dinv.astype(jnp.bfloat16)

    @pl.when(i == s_l1)
    def _():
        n = delta_ref.shape[1]
        hk = n // 2
        for k in range(2):
            sk = pl.ds(k * hk, hk)
            pts = p1t_ref[:, sk] * dinvb_ref[:, sk]
            part = jnp.dot(pts, adjb_ref[sk, :],
                           preferred_element_type=jnp.float32)
            if k == 0:
                acc_ref[...] = part
            else:
                acc_ref[...] += part
        w2t = w2_ref[...].T
        b1t = b1_ref[...].T
        for k in range(n // ob):
            sl = pl.ds(k * ob, ob)
            ptk = (p1t_ref[:, sl] * dinvb_ref[:, sl]).astype(jnp.float32)
            corr = delta_ref[:, sl] * ptk
            htk = jnp.maximum(
                deg_ref[:, sl] * (acc_ref[:, sl] + corr) + b1t,
                0.0).astype(jnp.bfloat16)
            p2t_ref[:, sl] = jnp.dot(w2t, htk,
                                     preferred_element_type=jnp.float32
                                     ).astype(jnp.bfloat16)

    @pl.when(i == s_l2)
    def _():
        n = delta_ref.shape[1]
        hk = n // 2
        for k in range(2):
            sk = pl.ds(k * hk, hk)
            pts = p2t_ref[:, sk] * dinvb_ref[:, sk]
            part = jnp.dot(pts, adjb_ref[sk, :],
                           preferred_element_type=jnp.float32)
            if k == 0:
                acc_ref[...] = part
            else:
                acc_ref[...] += part

    @pl.when(i >= s_w)
    def _():
        sl = pl.ds((i - s_w) * ob, ob)
        p2ts = (p2t_ref[:, sl] * dinvb_ref[:, sl]).astype(jnp.float32)
        out_t = deg_ref[:, sl] * (acc_ref[:, sl] + delta_ref[:, sl] * p2ts) \
            + b2_ref[...].T
        o_ref[...] = jnp.maximum(out_t, 0.0).T


_DB = 128
_OB = 512


def kernel(x, adj, W1, b1, W2, b2):
    n = adj.shape[0]
    f = x.shape[1]
    h1 = W1.shape[1]
    h2 = W2.shape[1]
    nb = n // _DB
    no = n // _OB
    nsteps = nb + 2 + no

    w1_b = W1.astype(jnp.bfloat16)
    w2_b = W2.astype(jnp.bfloat16)
    b1_c = b1.reshape(1, -1)
    b2_c = b2.reshape(1, -1)

    s_w = nb + 2

    out = pl.pallas_call(
        functools.partial(_gcn_kernel, nb=nb, db=_DB, ob=_OB),
        grid=(nsteps,),
        in_specs=[
            pl.BlockSpec((_DB, n), lambda i: (jnp.minimum(i, nb - 1), 0)),
            pl.BlockSpec((_DB, f), lambda i: (jnp.minimum(i, nb - 1), 0)),
            pl.BlockSpec((f, h1), lambda i: (0, 0)),
            pl.BlockSpec((h1, h2), lambda i: (0, 0)),
            pl.BlockSpec((1, h1), lambda i: (0, 0)),
            pl.BlockSpec((1, h2), lambda i: (0, 0)),
        ],
        out_specs=pl.BlockSpec(
            (_OB, h2), lambda i: (jnp.clip(i - s_w, 0, no - 1), 0)),
        out_shape=jax.ShapeDtypeStruct((n, h2), jnp.float32),
        scratch_shapes=[
            pltpu.VMEM((n, n), jnp.bfloat16),
            pltpu.VMEM((h1, n), jnp.bfloat16),
            pltpu.VMEM((h2, n), jnp.bfloat16),
            pltpu.VMEM((h1, n), jnp.float32),
            pltpu.VMEM((1, n), jnp.float32),
            pltpu.VMEM((1, n), jnp.bfloat16),
            pltpu.VMEM((1, n), jnp.float32),
        ],
        compiler_params=pltpu.CompilerParams(
            dimension_semantics=("arbitrary",)),
    )(adj, x, w1_b, w2_b, b1_c, b2_c)

    return out
